# Initial kernel scaffold; baseline (speedup 1.0000x reference)
import jax
import jax.numpy as jnp
from jax.experimental import pallas as pl


def kernel(x, edge_index, edge_weight, W, b):
    raise NotImplementedError("write your pallas kernel here")


# speedup vs baseline: 1.2828x; 1.2828x over previous
import jax
import jax.numpy as jnp
from jax import lax
from jax.experimental import pallas as pl
from jax.experimental.pallas import tpu as pltpu
from jax.experimental.pallas import tpu_sc as plsc

N_NODES = 10000
N_EDGES = 160000
D = 256

NC = 2
NS = 16
NT = NC * NS
LANES = 16
D_VECS = D // LANES

ROWS = 312
ROWS_LAST = N_NODES - ROWS * (NT - 1)
ACC_ROWS = ROWS_LAST

B = 80
QCAP = B + LANES
CE = 2000
N_ECH = N_EDGES // CE
GROUPS = CE // LANES


def _sc_body(x_hbm, src_hbm, dst_hbm, w_hbm, out_hbm,
             srcb, dstb, wb, qsrc, qw, qloc, rows_v, acc_v, sem):
    c = lax.axis_index("c")
    s = lax.axis_index("s")
    wid = s * NC + c
    lo = wid * ROWS
    n_rows = jnp.where(wid == NT - 1, ROWS_LAST, ROWS)
    hi = lo + n_rows

    zero16f = jnp.zeros((LANES,), jnp.float32)
    zero16i = jnp.zeros((LANES,), jnp.int32)

    def _zero_row(r, _):
        for d in range(D_VECS):
            acc_v[r, pl.ds(d * LANES, LANES)] = zero16f
        return 0
    lax.fori_loop(0, ACC_ROWS, _zero_row, 0)
    for i in range(QCAP // LANES):
        sl = pl.ds(i * LANES, LANES)
        qsrc[sl] = zero16i
        qloc[sl] = zero16i
        qw[sl] = zero16f

    def _fire_batch():
        pltpu.async_copy(x_hbm.at[qsrc.at[pl.ds(0, B)]], rows_v, sem).wait()

        def _acc_group(g, _):
            wvec = qw[pl.ds(g * LANES, LANES)]
            lvec = qloc[pl.ds(g * LANES, LANES)]
            for j in range(LANES):
                wv = wvec[j]
                lv = lvec[j]
                e = g * LANES + j
                for d in range(D_VECS):
                    sl = pl.ds(d * LANES, LANES)
                    acc_v[lv, sl] = acc_v[lv, sl] + rows_v[e, sl] * wv
            return 0
        lax.fori_loop(0, B // LANES, _acc_group, 0, unroll=False)

    def _chunk(kc, cnt):
        base = kc * CE
        pltpu.sync_copy(src_hbm.at[pl.ds(base, CE)], srcb)
        pltpu.sync_copy(dst_hbm.at[pl.ds(base, CE)], dstb)
        pltpu.sync_copy(w_hbm.at[pl.ds(base, CE)], wb)

        def _group(g, cnt):
            sl = pl.ds(g * LANES, LANES)
            d16 = dstb[sl]
            m = (d16 >= lo) & (d16 < hi)
            plsc.store_compressed(qsrc.at[pl.ds(cnt, LANES)], srcb[sl], mask=m)
            plsc.store_compressed(qw.at[pl.ds(cnt, LANES)], wb[sl], mask=m)
            plsc.store_compressed(qloc.at[pl.ds(cnt, LANES)], d16 - lo, mask=m)
            cnt = cnt + plsc.all_reduce_population_count(m)[0]

            full = cnt >= B

            @pl.when(full)
            def _():
                _fire_batch()
                qsrc[pl.ds(0, LANES)] = qsrc[pl.ds(B, LANES)]
                qw[pl.ds(0, LANES)] = qw[pl.ds(B, LANES)]
                qloc[pl.ds(0, LANES)] = qloc[pl.ds(B, LANES)]

            return jnp.where(full, cnt - B, cnt)

        return lax.fori_loop(0, GROUPS, _group, cnt, unroll=False)

    cnt = lax.fori_loop(0, N_ECH, _chunk, jnp.int32(0), unroll=False)

    lane = lax.iota(jnp.int32, LANES)
    for i in range(B // LANES):
        sl = pl.ds(i * LANES, LANES)
        valid = (lane + i * LANES) < cnt
        qsrc[sl] = jnp.where(valid, qsrc[sl], 0)
        qloc[sl] = jnp.where(valid, qloc[sl], 0)
        qw[sl] = jnp.where(valid, qw[sl], 0.0)

    @pl.when(cnt > 0)
    def _():
        _fire_batch()

    @pl.when(wid < NT - 1)
    def _():
        pltpu.sync_copy(acc_v.at[pl.ds(0, ROWS)], out_hbm.at[pl.ds(lo, ROWS)])

    @pl.when(wid == NT - 1)
    def _():
        pltpu.sync_copy(acc_v.at[pl.ds(0, ROWS_LAST)],
                        out_hbm.at[pl.ds(lo, ROWS_LAST)])


def _make_sc_aggregate():
    return pl.kernel(
        _sc_body,
        out_type=jax.ShapeDtypeStruct((N_NODES, D), jnp.float32),
        mesh=plsc.VectorSubcoreMesh(core_axis_name="c", subcore_axis_name="s"),
        compiler_params=pltpu.CompilerParams(needs_layout_passes=False),
        scratch_types=[
            pltpu.VMEM((CE,), jnp.int32),
            pltpu.VMEM((CE,), jnp.int32),
            pltpu.VMEM((CE,), jnp.float32),
            pltpu.VMEM((QCAP,), jnp.int32),
            pltpu.VMEM((QCAP,), jnp.float32),
            pltpu.VMEM((QCAP,), jnp.int32),
            pltpu.VMEM((B, D), jnp.float32),
            pltpu.VMEM((ACC_ROWS, D), jnp.float32),
            pltpu.SemaphoreType.DMA,
        ],
    )


def _mm_body(agg_ref, w_ref, b_ref, o_ref):
    acc = jnp.dot(agg_ref[...], w_ref[...], preferred_element_type=jnp.float32)
    o_ref[...] = jnp.maximum(acc + b_ref[...], 0.0)


BM = 400


def _mm_relu(agg, W, b):
    return pl.pallas_call(
        _mm_body,
        grid=(N_NODES // BM,),
        in_specs=[
            pl.BlockSpec((BM, D), lambda i: (i, 0)),
            pl.BlockSpec((D, D), lambda i: (0, 0)),
            pl.BlockSpec((1, D), lambda i: (0, 0)),
        ],
        out_specs=pl.BlockSpec((BM, D), lambda i: (i, 0)),
        out_shape=jax.ShapeDtypeStruct((N_NODES, D), jnp.float32),
    )(agg, W, b.reshape(1, D))


def kernel(x, edge_index, edge_weight, W, b):
    ei = edge_index.astype(jnp.int32)
    dst = ei[0]
    src = ei[1]
    agg = _make_sc_aggregate()(x, src, dst, edge_weight)
    return _mm_relu(agg, W, b)
